# Initial kernel scaffold; baseline (speedup 1.0000x reference)
import jax
import jax.numpy as jnp
from jax.experimental import pallas as pl


def kernel(q, mu, receivers, edge_indices, edge_weights, edge_versors, edge_attrs, W1, b1, W2, b2, Wf, bf, Wmix, Wm1, bm1, Wm2, bm2):
    raise NotImplementedError("write your pallas kernel here")


# speedup vs baseline: 5.3844x; 5.3844x over previous
import functools

import jax
import jax.numpy as jnp
from jax.experimental import pallas as pl
from jax.experimental.pallas import tpu as pltpu

N = 10000
E = 320000
C = 128
BD = 16
CUTOFF = 5.0
EPS = 1e-8

BN = 1000
BE = 4000


def _silu(x):
    return x * jax.nn.sigmoid(x)


def _node_mlp_body(q_ref, w1_ref, b1_ref, w2_ref, b2_ref, x_ref):
    h = _silu(jnp.dot(q_ref[...], w1_ref[...], preferred_element_type=jnp.float32)
              + b1_ref[...])
    x_ref[...] = (jnp.dot(h, w2_ref[...], preferred_element_type=jnp.float32)
                  + b2_ref[...])


def _node_mlp(q, W1, b1, W2, b2):
    grid = (N // BN,)
    return pl.pallas_call(
        _node_mlp_body,
        grid=grid,
        in_specs=[
            pl.BlockSpec((BN, C), lambda i: (i, 0)),
            pl.BlockSpec((C, C), lambda i: (0, 0)),
            pl.BlockSpec((1, C), lambda i: (0, 0)),
            pl.BlockSpec((C, 3 * C), lambda i: (0, 0)),
            pl.BlockSpec((1, 3 * C), lambda i: (0, 0)),
        ],
        out_specs=pl.BlockSpec((BN, 3 * C), lambda i: (i, 0)),
        out_shape=jax.ShapeDtypeStruct((N, 3 * C), jnp.float32),
    )(q, W1, b1.reshape(1, C), W2, b2.reshape(1, 3 * C))


def _edge_filter_body(ea_ref, ew_ref, wf_ref, bf_ref, wij_ref):
    w = (jnp.dot(ea_ref[...], wf_ref[...], preferred_element_type=jnp.float32)
         + bf_ref[...])
    ew = ew_ref[...]
    fcut = 0.5 * (jnp.cos(jnp.pi * ew / CUTOFF) + 1.0)
    fcut = fcut * (ew < CUTOFF).astype(jnp.float32)
    wij_ref[...] = w * fcut


def _edge_filter(edge_attrs, edge_weights, Wf, bf):
    grid = (E // BE,)
    return pl.pallas_call(
        _edge_filter_body,
        grid=grid,
        in_specs=[
            pl.BlockSpec((BE, BD), lambda i: (i, 0)),
            pl.BlockSpec((BE, 1), lambda i: (i, 0)),
            pl.BlockSpec((BD, 3 * C), lambda i: (0, 0)),
            pl.BlockSpec((1, 3 * C), lambda i: (0, 0)),
        ],
        out_specs=pl.BlockSpec((BE, 3 * C), lambda i: (i, 0)),
        out_shape=jax.ShapeDtypeStruct((E, 3 * C), jnp.float32),
    )(edge_attrs, edge_weights.reshape(E, 1), Wf, bf.reshape(1, 3 * C))


def _mixing_body(q_ref, mu0_ref, mu1_ref, mu2_ref,
                 dq_ref, dm0_ref, dm1_ref, dm2_ref,
                 wmix_ref, wm1_ref, bm1_ref, wm2_ref, bm2_ref,
                 qo_ref, mo0_ref, mo1_ref, mo2_ref):
    qq = q_ref[...] + dq_ref[...]
    mu2 = [mu0_ref[...] + dm0_ref[...],
           mu1_ref[...] + dm1_ref[...],
           mu2_ref[...] + dm2_ref[...]]
    wmix = wmix_ref[...]
    mix = [jnp.dot(m, wmix, preferred_element_type=jnp.float32) for m in mu2]
    muV = [m[:, :C] for m in mix]
    muW = [m[:, C:] for m in mix]
    muVn = jnp.sqrt(muV[0] * muV[0] + muV[1] * muV[1] + muV[2] * muV[2] + EPS)
    ctx = jnp.concatenate([qq, muVn], axis=1)
    h = _silu(jnp.dot(ctx, wm1_ref[...], preferred_element_type=jnp.float32)
              + bm1_ref[...])
    y = (jnp.dot(h, wm2_ref[...], preferred_element_type=jnp.float32)
         + bm2_ref[...])
    dq_i = y[:, :C]
    dmu_i = y[:, C:2 * C]
    dqmu_i = y[:, 2 * C:]
    s = muV[0] * muW[0] + muV[1] * muW[1] + muV[2] * muW[2]
    qo_ref[...] = qq + dq_i + dqmu_i * s
    mo0_ref[...] = mu2[0] + dmu_i * muW[0]
    mo1_ref[...] = mu2[1] + dmu_i * muW[1]
    mo2_ref[...] = mu2[2] + dmu_i * muW[2]


def _mixing(q, mu0, mu1, mu2, dq_agg, dm0, dm1, dm2, Wmix, Wm1, bm1, Wm2, bm2):
    grid = (N // BN,)
    node_spec = pl.BlockSpec((BN, C), lambda i: (i, 0))
    return pl.pallas_call(
        _mixing_body,
        grid=grid,
        in_specs=[
            node_spec, node_spec, node_spec, node_spec,
            node_spec, node_spec, node_spec, node_spec,
            pl.BlockSpec((C, 2 * C), lambda i: (0, 0)),
            pl.BlockSpec((2 * C, C), lambda i: (0, 0)),
            pl.BlockSpec((1, C), lambda i: (0, 0)),
            pl.BlockSpec((C, 3 * C), lambda i: (0, 0)),
            pl.BlockSpec((1, 3 * C), lambda i: (0, 0)),
        ],
        out_specs=[node_spec, node_spec, node_spec, node_spec],
        out_shape=[jax.ShapeDtypeStruct((N, C), jnp.float32)] * 4,
    )(q, mu0, mu1, mu2, dq_agg, dm0, dm1, dm2,
      Wmix, Wm1, bm1.reshape(1, C), Wm2, bm2.reshape(1, 3 * C))


def kernel(q, mu, receivers, edge_indices, edge_weights, edge_versors, edge_attrs,
           W1, b1, W2, b2, Wf, bf, Wmix, Wm1, bm1, Wm2, bm2):
    del receivers
    x = _node_mlp(q, W1, b1, W2, b2)
    wij = _edge_filter(edge_attrs, edge_weights, Wf, bf)

    idx_i = edge_indices[0]
    idx_j = edge_indices[1]
    mu_d = [mu[:, d, :] for d in range(3)]

    mm = wij * jnp.take(x, idx_j, axis=0)
    dq = mm[:, :C]
    m1 = mm[:, C:2 * C]
    m2 = mm[:, 2 * C:]
    dq_agg = jax.ops.segment_sum(dq, idx_i, num_segments=N)
    dmu_agg = []
    for d in range(3):
        msg = m1 * edge_versors[:, d:d + 1] + m2 * jnp.take(mu_d[d], idx_j, axis=0)
        dmu_agg.append(jax.ops.segment_sum(msg, idx_i, num_segments=N))

    qo, mo0, mo1, mo2 = _mixing(q, mu_d[0], mu_d[1], mu_d[2],
                                dq_agg, dmu_agg[0], dmu_agg[1], dmu_agg[2],
                                Wmix, Wm1, bm1, Wm2, bm2)
    return qo, jnp.stack([mo0, mo1, mo2], axis=1)


# speedup vs baseline: 5.5852x; 1.0373x over previous
import dataclasses
import functools

import jax
import jax.numpy as jnp
from jax.experimental import pallas as pl
from jax.experimental.pallas import tpu as pltpu
from jax.experimental.pallas import tpu_sc as plsc

N = 10000
E = 320000
C = 128
BD = 16
CUTOFF = 5.0
EPS = 1e-8

BN = 1000
BE = 4000
NW = 32
EPW = E // NW
BEK = 40
ZR = 40


def _silu(x):
    return x * jax.nn.sigmoid(x)


def _node_mlp_body(q_ref, w1_ref, b1_ref, w2_ref, b2_ref,
                   x0_ref, x1_ref, x2_ref):
    h = _silu(jnp.dot(q_ref[...], w1_ref[...], preferred_element_type=jnp.float32)
              + b1_ref[...])
    x = (jnp.dot(h, w2_ref[...], preferred_element_type=jnp.float32)
         + b2_ref[...])
    x0_ref[...] = x[:, :C]
    x1_ref[...] = x[:, C:2 * C]
    x2_ref[...] = x[:, 2 * C:]


def _node_mlp(q, W1, b1, W2, b2):
    node_spec = pl.BlockSpec((BN, C), lambda i: (i, 0))
    return pl.pallas_call(
        _node_mlp_body,
        grid=(N // BN,),
        in_specs=[
            pl.BlockSpec((BN, C), lambda i: (i, 0)),
            pl.BlockSpec((C, C), lambda i: (0, 0)),
            pl.BlockSpec((1, C), lambda i: (0, 0)),
            pl.BlockSpec((C, 3 * C), lambda i: (0, 0)),
            pl.BlockSpec((1, 3 * C), lambda i: (0, 0)),
        ],
        out_specs=[node_spec, node_spec, node_spec],
        out_shape=[jax.ShapeDtypeStruct((N, C), jnp.float32)] * 3,
    )(q, W1, b1.reshape(1, C), W2, b2.reshape(1, 3 * C))


def _edge_filter_body(ea_ref, ew_ref, wf_ref, bf_ref, w0_ref, w1_ref, w2_ref):
    w = (jnp.dot(ea_ref[...], wf_ref[...], preferred_element_type=jnp.float32)
         + bf_ref[...])
    ew = ew_ref[...]
    fcut = 0.5 * (jnp.cos(jnp.pi * ew / CUTOFF) + 1.0)
    fcut = fcut * (ew < CUTOFF).astype(jnp.float32)
    w = w * fcut
    w0_ref[...] = w[:, :C]
    w1_ref[...] = w[:, C:2 * C]
    w2_ref[...] = w[:, 2 * C:]


def _edge_filter(edge_attrs, edge_weights, Wf, bf):
    edge_spec = pl.BlockSpec((BE, C), lambda i: (i, 0))
    return pl.pallas_call(
        _edge_filter_body,
        grid=(E // BE,),
        in_specs=[
            pl.BlockSpec((BE, BD), lambda i: (i, 0)),
            pl.BlockSpec((BE, 1), lambda i: (i, 0)),
            pl.BlockSpec((BD, 3 * C), lambda i: (0, 0)),
            pl.BlockSpec((1, 3 * C), lambda i: (0, 0)),
        ],
        out_specs=[edge_spec, edge_spec, edge_spec],
        out_shape=[jax.ShapeDtypeStruct((E, C), jnp.float32)] * 3,
    )(edge_attrs, edge_weights.reshape(E, 1), Wf, bf.reshape(1, 3 * C))


def _sc_body(x0, x1, x2, w0, w1, w2, mu0, mu1, mu2, ii, jj, v0, v1, v2,
             dqp, dmup, m1h, m2h,
             acc, jv, iv, b_a, b_b, b_c, b_d, b_e, b_f, b_g, b_h, b_i,
             v_vmem):
    cid = jax.lax.axis_index("c")
    sid = jax.lax.axis_index("s")
    wid = cid * 16 + sid
    ebase = wid * EPW
    mus = [mu0, mu1, mu2]
    vs = [v0, v1, v2]

    def zero_acc():
        @pl.loop(0, ZR)
        def _(r):
            for cc in range(0, C, 16):
                b_g[r, pl.ds(cc, 16)] = jnp.zeros((16,), jnp.float32)

        @pl.loop(0, (N // ZR + 15) // 16)
        def _(k):
            idx = k * 16 + sid

            @pl.when(idx < N // ZR)
            def _():
                pltpu.sync_copy(b_g, acc.at[pl.ds(idx * ZR, ZR)])
        plsc.subcore_barrier()

    def flush(dst):
        plsc.subcore_barrier()

        @pl.when(sid == 0)
        def _():
            pltpu.sync_copy(acc, dst)
        plsc.subcore_barrier()

    zero_acc()

    @pl.loop(0, EPW, step=BEK)
    def _(eo):
        base = ebase + eo
        pltpu.sync_copy(jj.at[pl.ds(base, BEK)], jv)
        pltpu.sync_copy(ii.at[pl.ds(base, BEK)], iv)
        pltpu.sync_copy(x0.at[jv], b_a)
        pltpu.sync_copy(x1.at[jv], b_b)
        pltpu.sync_copy(x2.at[jv], b_c)
        pltpu.sync_copy(w0.at[pl.ds(base, BEK)], b_d)
        pltpu.sync_copy(w1.at[pl.ds(base, BEK)], b_e)
        pltpu.sync_copy(w2.at[pl.ds(base, BEK)], b_f)

        @pl.loop(0, BEK)
        def _(b):
            for cc in range(0, C, 16):
                sl = (b, pl.ds(cc, 16))
                b_g[sl] = b_d[sl] * b_a[sl]
                b_h[sl] = b_e[sl] * b_b[sl]
                b_i[sl] = b_f[sl] * b_c[sl]

        pltpu.sync_copy(b_h, m1h.at[pl.ds(base, BEK)])
        pltpu.sync_copy(b_i, m2h.at[pl.ds(base, BEK)])
        pltpu.sync_copy(b_g, acc.at[iv], add=True)

    flush(dqp.at[cid])

    for d in range(3):
        zero_acc()

        @pl.loop(0, EPW, step=BEK)
        def _(eo):
            base = ebase + eo
            pltpu.sync_copy(jj.at[pl.ds(base, BEK)], jv)
            pltpu.sync_copy(ii.at[pl.ds(base, BEK)], iv)
            pltpu.sync_copy(m1h.at[pl.ds(base, BEK)], b_a)
            pltpu.sync_copy(m2h.at[pl.ds(base, BEK)], b_b)
            pltpu.sync_copy(mus[d].at[jv], b_c)
            pltpu.sync_copy(vs[d].at[pl.ds(base, BEK)], v_vmem)

            @pl.loop(0, BEK)
            def _(b):
                b16 = jax.lax.broadcast(b, (16,))
                vv = plsc.load_gather(v_vmem, [b16])
                for cc in range(0, C, 16):
                    sl = (b, pl.ds(cc, 16))
                    b_g[sl] = b_a[sl] * vv + b_b[sl] * b_c[sl]

            pltpu.sync_copy(b_g, acc.at[iv], add=True)

        flush(dmup.at[d, cid])


def _sc_edge(x0, x1, x2, w0, w1, w2, mu0, mu1, mu2, ii, jj, v0, v1, v2):
    mesh = plsc.VectorSubcoreMesh(core_axis_name="c", subcore_axis_name="s")
    f32 = jnp.float32
    cp = pltpu.CompilerParams()
    if "needs_layout_passes" in pltpu.CompilerParams.__dataclass_fields__:
        cp = dataclasses.replace(cp, needs_layout_passes=False)
    run = pl.kernel(
        _sc_body,
        mesh=mesh,
        compiler_params=cp,
        out_type=[
            jax.ShapeDtypeStruct((2, N, C), f32),
            jax.ShapeDtypeStruct((3, 2, N, C), f32),
            jax.ShapeDtypeStruct((E, C), f32),
            jax.ShapeDtypeStruct((E, C), f32),
        ],
        scratch_types=[
            pltpu.VMEM_SHARED((N, C), f32),
            pltpu.VMEM((BEK,), jnp.int32),
            pltpu.VMEM((BEK,), jnp.int32),
        ] + [pltpu.VMEM((BEK, C), f32)] * 9 + [
            pltpu.VMEM((BEK,), f32),
        ],
    )
    return run(x0, x1, x2, w0, w1, w2, mu0, mu1, mu2, ii, jj, v0, v1, v2)


def _mixing_body(q_ref, mu0_ref, mu1_ref, mu2_ref,
                 dq0_ref, dq1_ref,
                 dm00_ref, dm01_ref, dm10_ref, dm11_ref, dm20_ref, dm21_ref,
                 wmix_ref, wm1_ref, bm1_ref, wm2_ref, bm2_ref,
                 qo_ref, mo0_ref, mo1_ref, mo2_ref):
    qq = q_ref[...] + dq0_ref[...] + dq1_ref[...]
    mu2 = [mu0_ref[...] + dm00_ref[...] + dm01_ref[...],
           mu1_ref[...] + dm10_ref[...] + dm11_ref[...],
           mu2_ref[...] + dm20_ref[...] + dm21_ref[...]]
    wmix = wmix_ref[...]
    mix = [jnp.dot(m, wmix, preferred_element_type=jnp.float32) for m in mu2]
    muV = [m[:, :C] for m in mix]
    muW = [m[:, C:] for m in mix]
    muVn = jnp.sqrt(muV[0] * muV[0] + muV[1] * muV[1] + muV[2] * muV[2] + EPS)
    ctx = jnp.concatenate([qq, muVn], axis=1)
    h = _silu(jnp.dot(ctx, wm1_ref[...], preferred_element_type=jnp.float32)
              + bm1_ref[...])
    y = (jnp.dot(h, wm2_ref[...], preferred_element_type=jnp.float32)
         + bm2_ref[...])
    dq_i = y[:, :C]
    dmu_i = y[:, C:2 * C]
    dqmu_i = y[:, 2 * C:]
    s = muV[0] * muW[0] + muV[1] * muW[1] + muV[2] * muW[2]
    qo_ref[...] = qq + dq_i + dqmu_i * s
    mo0_ref[...] = mu2[0] + dmu_i * muW[0]
    mo1_ref[...] = mu2[1] + dmu_i * muW[1]
    mo2_ref[...] = mu2[2] + dmu_i * muW[2]


def _mixing(q, mu0, mu1, mu2, dq0, dq1, dm00, dm01, dm10, dm11, dm20, dm21,
            Wmix, Wm1, bm1, Wm2, bm2):
    node_spec = pl.BlockSpec((BN, C), lambda i: (i, 0))
    return pl.pallas_call(
        _mixing_body,
        grid=(N // BN,),
        in_specs=[node_spec] * 12 + [
            pl.BlockSpec((C, 2 * C), lambda i: (0, 0)),
            pl.BlockSpec((2 * C, C), lambda i: (0, 0)),
            pl.BlockSpec((1, C), lambda i: (0, 0)),
            pl.BlockSpec((C, 3 * C), lambda i: (0, 0)),
            pl.BlockSpec((1, 3 * C), lambda i: (0, 0)),
        ],
        out_specs=[node_spec, node_spec, node_spec, node_spec],
        out_shape=[jax.ShapeDtypeStruct((N, C), jnp.float32)] * 4,
    )(q, mu0, mu1, mu2, dq0, dq1, dm00, dm01, dm10, dm11, dm20, dm21,
      Wmix, Wm1, bm1.reshape(1, C), Wm2, bm2.reshape(1, 3 * C))


def kernel(q, mu, receivers, edge_indices, edge_weights, edge_versors, edge_attrs,
           W1, b1, W2, b2, Wf, bf, Wmix, Wm1, bm1, Wm2, bm2):
    del receivers
    x0, x1, x2 = _node_mlp(q, W1, b1, W2, b2)
    w0, w1, w2 = _edge_filter(edge_attrs, edge_weights, Wf, bf)

    idx_i = edge_indices[0]
    idx_j = edge_indices[1]
    mu_d = [mu[:, d, :] for d in range(3)]
    v_d = [edge_versors[:, d] for d in range(3)]

    dqp, dmup, _m1, _m2 = _sc_edge(x0, x1, x2, w0, w1, w2,
                                   mu_d[0], mu_d[1], mu_d[2],
                                   idx_i, idx_j, v_d[0], v_d[1], v_d[2])

    qo, mo0, mo1, mo2 = _mixing(
        q, mu_d[0], mu_d[1], mu_d[2],
        dqp[0], dqp[1],
        dmup[0, 0], dmup[0, 1], dmup[1, 0], dmup[1, 1], dmup[2, 0], dmup[2, 1],
        Wmix, Wm1, bm1, Wm2, bm2)
    return qo, jnp.stack([mo0, mo1, mo2], axis=1)


# speedup vs baseline: 8.1389x; 1.4572x over previous
import dataclasses
import functools

import jax
import jax.numpy as jnp
from jax.experimental import pallas as pl
from jax.experimental.pallas import tpu as pltpu
from jax.experimental.pallas import tpu_sc as plsc

N = 10000
E = 320000
C = 128
BD = 16
CUTOFF = 5.0
EPS = 1e-8

BN = 1000
BE = 4000
NW = 32
EPW = E // NW
BEK = 40
ZR = 40


def _silu(x):
    return x * jax.nn.sigmoid(x)


def _node_mlp_body(q_ref, w1_ref, b1_ref, w2_ref, b2_ref, x_ref):
    h = _silu(jnp.dot(q_ref[...], w1_ref[...], preferred_element_type=jnp.float32)
              + b1_ref[...])
    x_ref[...] = (jnp.dot(h, w2_ref[...], preferred_element_type=jnp.float32)
                  + b2_ref[...])


def _node_mlp(q, W1, b1, W2, b2):
    return pl.pallas_call(
        _node_mlp_body,
        grid=(N // BN,),
        in_specs=[
            pl.BlockSpec((BN, C), lambda i: (i, 0)),
            pl.BlockSpec((C, C), lambda i: (0, 0)),
            pl.BlockSpec((1, C), lambda i: (0, 0)),
            pl.BlockSpec((C, 3 * C), lambda i: (0, 0)),
            pl.BlockSpec((1, 3 * C), lambda i: (0, 0)),
        ],
        out_specs=pl.BlockSpec((BN, 3 * C), lambda i: (i, 0)),
        out_shape=jax.ShapeDtypeStruct((N, 3 * C), jnp.float32),
    )(q, W1, b1.reshape(1, C), W2, b2.reshape(1, 3 * C))


def _edge_filter_body(ea_ref, ew_ref, wf_ref, bf_ref, wij_ref):
    w = (jnp.dot(ea_ref[...], wf_ref[...], preferred_element_type=jnp.float32)
         + bf_ref[...])
    ew = ew_ref[...]
    fcut = 0.5 * (jnp.cos(jnp.pi * ew / CUTOFF) + 1.0)
    fcut = fcut * (ew < CUTOFF).astype(jnp.float32)
    wij_ref[...] = w * fcut


def _edge_filter(edge_attrs, edge_weights, Wf, bf):
    return pl.pallas_call(
        _edge_filter_body,
        grid=(E // BE,),
        in_specs=[
            pl.BlockSpec((BE, BD), lambda i: (i, 0)),
            pl.BlockSpec((BE, 1), lambda i: (i, 0)),
            pl.BlockSpec((BD, 3 * C), lambda i: (0, 0)),
            pl.BlockSpec((1, 3 * C), lambda i: (0, 0)),
        ],
        out_specs=pl.BlockSpec((BE, 3 * C), lambda i: (i, 0)),
        out_shape=jax.ShapeDtypeStruct((E, 3 * C), jnp.float32),
    )(edge_attrs, edge_weights.reshape(E, 1), Wf, bf.reshape(1, 3 * C))


def _sc_body(x, wij, mu0, mu1, mu2, ii, jj, v0, v1, v2,
             dqp, dmup, m12h,
             acc, jv, iv, xg, wb, msg, mA, mud, v_vmem, semA, semB):
    cid = jax.lax.axis_index("c")
    sid = jax.lax.axis_index("s")
    wid = cid * 16 + sid
    ebase = wid * EPW
    mus = [mu0, mu1, mu2]
    vs = [v0, v1, v2]

    def zero_acc():
        @pl.loop(0, ZR)
        def _(r):
            for cc in range(0, C, 16):
                msg[r, pl.ds(cc, 16)] = jnp.zeros((16,), jnp.float32)

        @pl.loop(0, (N // ZR + 15) // 16)
        def _(k):
            idx = k * 16 + sid

            @pl.when(idx < N // ZR)
            def _():
                pltpu.sync_copy(msg, acc.at[pl.ds(idx * ZR, ZR)])
        plsc.subcore_barrier()

    def flush(dst):
        plsc.subcore_barrier()

        @pl.when(sid == 0)
        def _():
            pltpu.sync_copy(acc, dst)
        plsc.subcore_barrier()

    zero_acc()

    @pl.loop(0, EPW, step=BEK)
    def _(eo):
        base = ebase + eo
        c1 = pltpu.async_copy(jj.at[pl.ds(base, BEK)], jv, semA)
        c2 = pltpu.async_copy(ii.at[pl.ds(base, BEK)], iv, semB)
        c1.wait()
        c2.wait()
        c3 = pltpu.async_copy(x.at[jv], xg, semA)
        c4 = pltpu.async_copy(wij.at[pl.ds(base, BEK)], wb, semB)
        c3.wait()
        c4.wait()

        @pl.loop(0, BEK)
        def _(b):
            for cc in range(0, C, 16):
                s0 = (b, pl.ds(cc, 16))
                s1 = (b, pl.ds(C + cc, 16))
                s2 = (b, pl.ds(2 * C + cc, 16))
                msg[s0] = wb[s0] * xg[s0]
                xg[s1] = wb[s1] * xg[s1]
                xg[s2] = wb[s2] * xg[s2]

        c5 = pltpu.async_copy(xg.at[pl.ds(0, BEK), pl.ds(C, C)],
                              m12h.at[pl.ds(base, BEK), pl.ds(0, C)], semA)
        c6 = pltpu.async_copy(xg.at[pl.ds(0, BEK), pl.ds(2 * C, C)],
                              m12h.at[pl.ds(base, BEK), pl.ds(C, C)], semB)
        pltpu.sync_copy(msg, acc.at[iv], add=True)
        c5.wait()
        c6.wait()

    flush(dqp.at[cid])

    for d in range(3):
        zero_acc()

        @pl.loop(0, EPW, step=BEK)
        def _(eo):
            base = ebase + eo
            c1 = pltpu.async_copy(jj.at[pl.ds(base, BEK)], jv, semA)
            c2 = pltpu.async_copy(ii.at[pl.ds(base, BEK)], iv, semB)
            c1.wait()
            c2.wait()
            c3 = pltpu.async_copy(mus[d].at[jv], mud, semA)
            c4 = pltpu.async_copy(m12h.at[pl.ds(base, BEK), pl.ds(0, C)],
                                  mA, semB)
            c5 = pltpu.async_copy(m12h.at[pl.ds(base, BEK), pl.ds(C, C)],
                                  msg, semA)
            c6 = pltpu.async_copy(vs[d].at[pl.ds(base, BEK)], v_vmem, semB)
            c3.wait()
            c4.wait()
            c5.wait()
            c6.wait()

            @pl.loop(0, BEK)
            def _(b):
                b16 = jax.lax.broadcast(b, (16,))
                vv = plsc.load_gather(v_vmem, [b16])
                for cc in range(0, C, 16):
                    sl = (b, pl.ds(cc, 16))
                    msg[sl] = mA[sl] * vv + msg[sl] * mud[sl]

            pltpu.sync_copy(msg, acc.at[iv], add=True)

        flush(dmup.at[d, cid])


def _sc_edge(x, wij, mu0, mu1, mu2, ii, jj, v0, v1, v2):
    mesh = plsc.VectorSubcoreMesh(core_axis_name="c", subcore_axis_name="s")
    f32 = jnp.float32
    cp = pltpu.CompilerParams()
    if "needs_layout_passes" in pltpu.CompilerParams.__dataclass_fields__:
        cp = dataclasses.replace(cp, needs_layout_passes=False)
    run = pl.kernel(
        _sc_body,
        mesh=mesh,
        compiler_params=cp,
        out_type=[
            jax.ShapeDtypeStruct((2, N, C), f32),
            jax.ShapeDtypeStruct((3, 2, N, C), f32),
            jax.ShapeDtypeStruct((E, 2 * C), f32),
        ],
        scratch_types=[
            pltpu.VMEM_SHARED((N, C), f32),
            pltpu.VMEM((BEK,), jnp.int32),
            pltpu.VMEM((BEK,), jnp.int32),
            pltpu.VMEM((BEK, 3 * C), f32),
            pltpu.VMEM((BEK, 3 * C), f32),
            pltpu.VMEM((BEK, C), f32),
            pltpu.VMEM((BEK, C), f32),
            pltpu.VMEM((BEK, C), f32),
            pltpu.VMEM((BEK,), f32),
            pltpu.SemaphoreType.DMA,
            pltpu.SemaphoreType.DMA,
        ],
    )
    return run(x, wij, mu0, mu1, mu2, ii, jj, v0, v1, v2)


def _mixing_body(q_ref, mu0_ref, mu1_ref, mu2_ref,
                 dq0_ref, dq1_ref,
                 dm00_ref, dm01_ref, dm10_ref, dm11_ref, dm20_ref, dm21_ref,
                 wmix_ref, wm1_ref, bm1_ref, wm2_ref, bm2_ref,
                 qo_ref, mo0_ref, mo1_ref, mo2_ref):
    qq = q_ref[...] + dq0_ref[...] + dq1_ref[...]
    mu2 = [mu0_ref[...] + dm00_ref[...] + dm01_ref[...],
           mu1_ref[...] + dm10_ref[...] + dm11_ref[...],
           mu2_ref[...] + dm20_ref[...] + dm21_ref[...]]
    wmix = wmix_ref[...]
    mix = [jnp.dot(m, wmix, preferred_element_type=jnp.float32) for m in mu2]
    muV = [m[:, :C] for m in mix]
    muW = [m[:, C:] for m in mix]
    muVn = jnp.sqrt(muV[0] * muV[0] + muV[1] * muV[1] + muV[2] * muV[2] + EPS)
    ctx = jnp.concatenate([qq, muVn], axis=1)
    h = _silu(jnp.dot(ctx, wm1_ref[...], preferred_element_type=jnp.float32)
              + bm1_ref[...])
    y = (jnp.dot(h, wm2_ref[...], preferred_element_type=jnp.float32)
         + bm2_ref[...])
    dq_i = y[:, :C]
    dmu_i = y[:, C:2 * C]
    dqmu_i = y[:, 2 * C:]
    s = muV[0] * muW[0] + muV[1] * muW[1] + muV[2] * muW[2]
    qo_ref[...] = qq + dq_i + dqmu_i * s
    mo0_ref[...] = mu2[0] + dmu_i * muW[0]
    mo1_ref[...] = mu2[1] + dmu_i * muW[1]
    mo2_ref[...] = mu2[2] + dmu_i * muW[2]


def _mixing(q, mu0, mu1, mu2, dq0, dq1, dm00, dm01, dm10, dm11, dm20, dm21,
            Wmix, Wm1, bm1, Wm2, bm2):
    node_spec = pl.BlockSpec((BN, C), lambda i: (i, 0))
    return pl.pallas_call(
        _mixing_body,
        grid=(N // BN,),
        in_specs=[node_spec] * 12 + [
            pl.BlockSpec((C, 2 * C), lambda i: (0, 0)),
            pl.BlockSpec((2 * C, C), lambda i: (0, 0)),
            pl.BlockSpec((1, C), lambda i: (0, 0)),
            pl.BlockSpec((C, 3 * C), lambda i: (0, 0)),
            pl.BlockSpec((1, 3 * C), lambda i: (0, 0)),
        ],
        out_specs=[node_spec, node_spec, node_spec, node_spec],
        out_shape=[jax.ShapeDtypeStruct((N, C), jnp.float32)] * 4,
    )(q, mu0, mu1, mu2, dq0, dq1, dm00, dm01, dm10, dm11, dm20, dm21,
      Wmix, Wm1, bm1.reshape(1, C), Wm2, bm2.reshape(1, 3 * C))


def kernel(q, mu, receivers, edge_indices, edge_weights, edge_versors, edge_attrs,
           W1, b1, W2, b2, Wf, bf, Wmix, Wm1, bm1, Wm2, bm2):
    del receivers
    x = _node_mlp(q, W1, b1, W2, b2)
    wij = _edge_filter(edge_attrs, edge_weights, Wf, bf)

    idx_i = edge_indices[0]
    idx_j = edge_indices[1]
    mu_d = [mu[:, d, :] for d in range(3)]
    v_d = [edge_versors[:, d] for d in range(3)]

    dqp, dmup, _m12 = _sc_edge(x, wij, mu_d[0], mu_d[1], mu_d[2],
                               idx_i, idx_j, v_d[0], v_d[1], v_d[2])

    qo, mo0, mo1, mo2 = _mixing(
        q, mu_d[0], mu_d[1], mu_d[2],
        dqp[0], dqp[1],
        dmup[0, 0], dmup[0, 1], dmup[1, 0], dmup[1, 1], dmup[2, 0], dmup[2, 1],
        Wmix, Wm1, bm1, Wm2, bm2)
    return qo, jnp.stack([mo0, mo1, mo2], axis=1)


# speedup vs baseline: 8.8505x; 1.0874x over previous
import dataclasses
import functools

import jax
import jax.numpy as jnp
from jax.experimental import pallas as pl
from jax.experimental.pallas import tpu as pltpu
from jax.experimental.pallas import tpu_sc as plsc

N = 10000
E = 320000
C = 128
BD = 16
CUTOFF = 5.0
EPS = 1e-8

BN = 1000
BE = 4000
NW = 32
EPW = E // NW
BEK = 40
ZR = 40


def _silu(x):
    return x * jax.nn.sigmoid(x)


def _node_mlp_body(q_ref, w1_ref, b1_ref, w2_ref, b2_ref, x_ref):
    h = _silu(jnp.dot(q_ref[...], w1_ref[...], preferred_element_type=jnp.float32)
              + b1_ref[...])
    x_ref[...] = (jnp.dot(h, w2_ref[...], preferred_element_type=jnp.float32)
                  + b2_ref[...])


def _node_mlp(q, W1, b1, W2, b2):
    return pl.pallas_call(
        _node_mlp_body,
        grid=(N // BN,),
        in_specs=[
            pl.BlockSpec((BN, C), lambda i: (i, 0)),
            pl.BlockSpec((C, C), lambda i: (0, 0)),
            pl.BlockSpec((1, C), lambda i: (0, 0)),
            pl.BlockSpec((C, 3 * C), lambda i: (0, 0)),
            pl.BlockSpec((1, 3 * C), lambda i: (0, 0)),
        ],
        out_specs=pl.BlockSpec((BN, 3 * C), lambda i: (i, 0)),
        out_shape=jax.ShapeDtypeStruct((N, 3 * C), jnp.float32),
    )(q, W1, b1.reshape(1, C), W2, b2.reshape(1, 3 * C))


def _edge_filter_body(ea_ref, ew_ref, wf_ref, bf_ref, wij_ref):
    w = (jnp.dot(ea_ref[...], wf_ref[...], preferred_element_type=jnp.float32)
         + bf_ref[...])
    ew = ew_ref[...]
    fcut = 0.5 * (jnp.cos(jnp.pi * ew / CUTOFF) + 1.0)
    fcut = fcut * (ew < CUTOFF).astype(jnp.float32)
    wij_ref[...] = w * fcut


def _edge_filter(edge_attrs, edge_weights, Wf, bf):
    return pl.pallas_call(
        _edge_filter_body,
        grid=(E // BE,),
        in_specs=[
            pl.BlockSpec((BE, BD), lambda i: (i, 0)),
            pl.BlockSpec((BE, 1), lambda i: (i, 0)),
            pl.BlockSpec((BD, 3 * C), lambda i: (0, 0)),
            pl.BlockSpec((1, 3 * C), lambda i: (0, 0)),
        ],
        out_specs=pl.BlockSpec((BE, 3 * C), lambda i: (i, 0)),
        out_shape=jax.ShapeDtypeStruct((E, 3 * C), jnp.float32),
    )(edge_attrs, edge_weights.reshape(E, 1), Wf, bf.reshape(1, 3 * C))


def _sc_body(x, wij, mu0, mu1, mu2, ii, jj, v0, v1, v2,
             dqp, dmup, m12h,
             acc, jv0, iv0, jv1, iv1, xg, wb, msg, mA, mud, v_vmem,
             semA, semB, semI, semO):
    cid = jax.lax.axis_index("c")
    sid = jax.lax.axis_index("s")
    wid = cid * 16 + sid
    ebase = wid * EPW
    mus = [mu0, mu1, mu2]
    vs = [v0, v1, v2]

    def zero_acc():
        @pl.loop(0, ZR)
        def _(r):
            for cc in range(0, C, 16):
                msg[r, pl.ds(cc, 16)] = jnp.zeros((16,), jnp.float32)

        @pl.loop(0, (N // ZR + 15) // 16)
        def _(k):
            idx = k * 16 + sid

            @pl.when(idx < N // ZR)
            def _():
                pltpu.sync_copy(msg, acc.at[pl.ds(idx * ZR, ZR)])
        plsc.subcore_barrier()

    def flush(dst):
        plsc.subcore_barrier()

        @pl.when(sid == 0)
        def _():
            pltpu.sync_copy(acc, dst)
        plsc.subcore_barrier()

    def fetch_idx(base, jdst, idst):
        cj = pltpu.async_copy(jj.at[pl.ds(base, BEK)], jdst, semI)
        ci = pltpu.async_copy(ii.at[pl.ds(base, BEK)], idst, semI)
        return cj, ci

    zero_acc()

    p0 = fetch_idx(ebase, jv0, iv0)
    p0[0].wait()
    p0[1].wait()

    @pl.loop(0, EPW, step=2 * BEK)
    def _(eo):
        def block(base, nxt_base, jvk, ivk, jvn, ivn, m12_prev):
            c3 = pltpu.async_copy(x.at[jvk], xg, semA)
            c4 = pltpu.async_copy(wij.at[pl.ds(base, BEK)], wb, semB)
            pn = fetch_idx(nxt_base, jvn, ivn)
            c3.wait()
            c4.wait()

            @pl.loop(0, BEK)
            def _(b):
                for cc in range(0, C, 16):
                    s0 = (b, pl.ds(cc, 16))
                    s1 = (b, pl.ds(C + cc, 16))
                    s2 = (b, pl.ds(2 * C + cc, 16))
                    msg[s0] = wb[s0] * xg[s0]
                    xg[s1] = wb[s1] * xg[s1]
                    xg[s2] = wb[s2] * xg[s2]

            c5 = pltpu.async_copy(xg.at[pl.ds(0, BEK), pl.ds(C, C)],
                                  m12h.at[pl.ds(base, BEK), pl.ds(0, C)], semO)
            c6 = pltpu.async_copy(xg.at[pl.ds(0, BEK), pl.ds(2 * C, C)],
                                  m12h.at[pl.ds(base, BEK), pl.ds(C, C)], semO)
            pltpu.sync_copy(msg, acc.at[ivk], add=True)
            pn[0].wait()
            pn[1].wait()
            return c5, c6

        baseA = ebase + eo
        baseB = baseA + BEK
        nxtA = jnp.minimum(baseA + 2 * BEK, ebase + EPW - BEK)
        mA_w = block(baseA, baseB, jv0, iv0, jv1, iv1, None)
        mA_w[0].wait()
        mA_w[1].wait()
        mB_w = block(baseB, nxtA, jv1, iv1, jv0, iv0, None)
        mB_w[0].wait()
        mB_w[1].wait()

    flush(dqp.at[cid])

    for d in range(3):
        zero_acc()

        pd = fetch_idx(ebase, jv0, iv0)
        pd[0].wait()
        pd[1].wait()

        @pl.loop(0, EPW, step=2 * BEK)
        def _(eo):
            def block(base, nxt_base, jvk, ivk, jvn, ivn):
                c3 = pltpu.async_copy(mus[d].at[jvk], mud, semA)
                c4 = pltpu.async_copy(m12h.at[pl.ds(base, BEK), pl.ds(0, C)],
                                      mA, semB)
                c5 = pltpu.async_copy(m12h.at[pl.ds(base, BEK), pl.ds(C, C)],
                                      msg, semB)
                c6 = pltpu.async_copy(vs[d].at[pl.ds(base, BEK)], v_vmem, semB)
                pn = fetch_idx(nxt_base, jvn, ivn)
                c3.wait()
                c4.wait()
                c5.wait()
                c6.wait()

                @pl.loop(0, BEK)
                def _(b):
                    b16 = jax.lax.broadcast(b, (16,))
                    vv = plsc.load_gather(v_vmem, [b16])
                    for cc in range(0, C, 16):
                        sl = (b, pl.ds(cc, 16))
                        msg[sl] = mA[sl] * vv + msg[sl] * mud[sl]

                pltpu.sync_copy(msg, acc.at[ivk], add=True)
                pn[0].wait()
                pn[1].wait()

            baseA = ebase + eo
            baseB = baseA + BEK
            nxtA = jnp.minimum(baseA + 2 * BEK, ebase + EPW - BEK)
            block(baseA, baseB, jv0, iv0, jv1, iv1)
            block(baseB, nxtA, jv1, iv1, jv0, iv0)

        flush(dmup.at[d, cid])


def _sc_edge(x, wij, mu0, mu1, mu2, ii, jj, v0, v1, v2):
    mesh = plsc.VectorSubcoreMesh(core_axis_name="c", subcore_axis_name="s")
    f32 = jnp.float32
    cp = pltpu.CompilerParams()
    if "needs_layout_passes" in pltpu.CompilerParams.__dataclass_fields__:
        cp = dataclasses.replace(cp, needs_layout_passes=False)
    run = pl.kernel(
        _sc_body,
        mesh=mesh,
        compiler_params=cp,
        out_type=[
            jax.ShapeDtypeStruct((2, N, C), f32),
            jax.ShapeDtypeStruct((3, 2, N, C), f32),
            jax.ShapeDtypeStruct((E, 2 * C), f32),
        ],
        scratch_types=[
            pltpu.VMEM_SHARED((N, C), f32),
            pltpu.VMEM((BEK,), jnp.int32),
            pltpu.VMEM((BEK,), jnp.int32),
            pltpu.VMEM((BEK,), jnp.int32),
            pltpu.VMEM((BEK,), jnp.int32),
            pltpu.VMEM((BEK, 3 * C), f32),
            pltpu.VMEM((BEK, 3 * C), f32),
            pltpu.VMEM((BEK, C), f32),
            pltpu.VMEM((BEK, C), f32),
            pltpu.VMEM((BEK, C), f32),
            pltpu.VMEM((BEK,), f32),
            pltpu.SemaphoreType.DMA,
            pltpu.SemaphoreType.DMA,
            pltpu.SemaphoreType.DMA,
            pltpu.SemaphoreType.DMA,
        ],
    )
    return run(x, wij, mu0, mu1, mu2, ii, jj, v0, v1, v2)


def _mixing_body(q_ref, mu0_ref, mu1_ref, mu2_ref,
                 dq0_ref, dq1_ref,
                 dm00_ref, dm01_ref, dm10_ref, dm11_ref, dm20_ref, dm21_ref,
                 wmix_ref, wm1_ref, bm1_ref, wm2_ref, bm2_ref,
                 qo_ref, mo0_ref, mo1_ref, mo2_ref):
    qq = q_ref[...] + dq0_ref[...] + dq1_ref[...]
    mu2 = [mu0_ref[...] + dm00_ref[...] + dm01_ref[...],
           mu1_ref[...] + dm10_ref[...] + dm11_ref[...],
           mu2_ref[...] + dm20_ref[...] + dm21_ref[...]]
    wmix = wmix_ref[...]
    mix = [jnp.dot(m, wmix, preferred_element_type=jnp.float32) for m in mu2]
    muV = [m[:, :C] for m in mix]
    muW = [m[:, C:] for m in mix]
    muVn = jnp.sqrt(muV[0] * muV[0] + muV[1] * muV[1] + muV[2] * muV[2] + EPS)
    ctx = jnp.concatenate([qq, muVn], axis=1)
    h = _silu(jnp.dot(ctx, wm1_ref[...], preferred_element_type=jnp.float32)
              + bm1_ref[...])
    y = (jnp.dot(h, wm2_ref[...], preferred_element_type=jnp.float32)
         + bm2_ref[...])
    dq_i = y[:, :C]
    dmu_i = y[:, C:2 * C]
    dqmu_i = y[:, 2 * C:]
    s = muV[0] * muW[0] + muV[1] * muW[1] + muV[2] * muW[2]
    qo_ref[...] = qq + dq_i + dqmu_i * s
    mo0_ref[...] = mu2[0] + dmu_i * muW[0]
    mo1_ref[...] = mu2[1] + dmu_i * muW[1]
    mo2_ref[...] = mu2[2] + dmu_i * muW[2]


def _mixing(q, mu0, mu1, mu2, dq0, dq1, dm00, dm01, dm10, dm11, dm20, dm21,
            Wmix, Wm1, bm1, Wm2, bm2):
    node_spec = pl.BlockSpec((BN, C), lambda i: (i, 0))
    return pl.pallas_call(
        _mixing_body,
        grid=(N // BN,),
        in_specs=[node_spec] * 12 + [
            pl.BlockSpec((C, 2 * C), lambda i: (0, 0)),
            pl.BlockSpec((2 * C, C), lambda i: (0, 0)),
            pl.BlockSpec((1, C), lambda i: (0, 0)),
            pl.BlockSpec((C, 3 * C), lambda i: (0, 0)),
            pl.BlockSpec((1, 3 * C), lambda i: (0, 0)),
        ],
        out_specs=[node_spec, node_spec, node_spec, node_spec],
        out_shape=[jax.ShapeDtypeStruct((N, C), jnp.float32)] * 4,
    )(q, mu0, mu1, mu2, dq0, dq1, dm00, dm01, dm10, dm11, dm20, dm21,
      Wmix, Wm1, bm1.reshape(1, C), Wm2, bm2.reshape(1, 3 * C))


def kernel(q, mu, receivers, edge_indices, edge_weights, edge_versors, edge_attrs,
           W1, b1, W2, b2, Wf, bf, Wmix, Wm1, bm1, Wm2, bm2):
    del receivers
    x = _node_mlp(q, W1, b1, W2, b2)
    wij = _edge_filter(edge_attrs, edge_weights, Wf, bf)

    idx_i = edge_indices[0]
    idx_j = edge_indices[1]
    mu_d = [mu[:, d, :] for d in range(3)]
    v_d = [edge_versors[:, d] for d in range(3)]

    dqp, dmup, _m12 = _sc_edge(x, wij, mu_d[0], mu_d[1], mu_d[2],
                               idx_i, idx_j, v_d[0], v_d[1], v_d[2])

    qo, mo0, mo1, mo2 = _mixing(
        q, mu_d[0], mu_d[1], mu_d[2],
        dqp[0], dqp[1],
        dmup[0, 0], dmup[0, 1], dmup[1, 0], dmup[1, 1], dmup[2, 0], dmup[2, 1],
        Wmix, Wm1, bm1, Wm2, bm2)
    return qo, jnp.stack([mo0, mo1, mo2], axis=1)


# speedup vs baseline: 11.3467x; 1.2820x over previous
import dataclasses
import functools

import jax
import jax.numpy as jnp
from jax.experimental import pallas as pl
from jax.experimental.pallas import tpu as pltpu
from jax.experimental.pallas import tpu_sc as plsc

N = 10000
E = 320000
C = 128
BD = 16
CUTOFF = 5.0
EPS = 1e-8

BN = 1000
BE = 4000
NW = 32
EPW = E // NW
BEK = 40
ZR = 40


def _silu(x):
    return x * jax.nn.sigmoid(x)


def _node_mlp_body(q_ref, w1_ref, b1_ref, w2_ref, b2_ref, x_ref):
    h = _silu(jnp.dot(q_ref[...], w1_ref[...], preferred_element_type=jnp.float32)
              + b1_ref[...])
    x_ref[...] = (jnp.dot(h, w2_ref[...], preferred_element_type=jnp.float32)
                  + b2_ref[...])


def _node_mlp(q, W1, b1, W2, b2):
    return pl.pallas_call(
        _node_mlp_body,
        grid=(N // BN,),
        in_specs=[
            pl.BlockSpec((BN, C), lambda i: (i, 0)),
            pl.BlockSpec((C, C), lambda i: (0, 0)),
            pl.BlockSpec((1, C), lambda i: (0, 0)),
            pl.BlockSpec((C, 3 * C), lambda i: (0, 0)),
            pl.BlockSpec((1, 3 * C), lambda i: (0, 0)),
        ],
        out_specs=pl.BlockSpec((BN, 3 * C), lambda i: (i, 0)),
        out_shape=jax.ShapeDtypeStruct((N, 3 * C), jnp.float32),
    )(q, W1, b1.reshape(1, C), W2, b2.reshape(1, 3 * C))


def _edge_filter_body(ea_ref, ew_ref, wf_ref, bf_ref, wij_ref):
    w = (jnp.dot(ea_ref[...], wf_ref[...], preferred_element_type=jnp.float32)
         + bf_ref[...])
    ew = ew_ref[...]
    fcut = 0.5 * (jnp.cos(jnp.pi * ew / CUTOFF) + 1.0)
    fcut = fcut * (ew < CUTOFF).astype(jnp.float32)
    wij_ref[...] = w * fcut


def _edge_filter(edge_attrs, edge_weights, Wf, bf):
    return pl.pallas_call(
        _edge_filter_body,
        grid=(E // BE,),
        in_specs=[
            pl.BlockSpec((BE, BD), lambda i: (i, 0)),
            pl.BlockSpec((BE, 1), lambda i: (i, 0)),
            pl.BlockSpec((BD, 3 * C), lambda i: (0, 0)),
            pl.BlockSpec((1, 3 * C), lambda i: (0, 0)),
        ],
        out_specs=pl.BlockSpec((BE, 3 * C), lambda i: (i, 0)),
        out_shape=jax.ShapeDtypeStruct((E, 3 * C), jnp.float32),
    )(edge_attrs, edge_weights.reshape(E, 1), Wf, bf.reshape(1, 3 * C))


def _sc_body(x, wij, mu0, mu1, mu2, ii, jj, v0, v1, v2,
             dqp, dmup, m12h,
             acc, jv0, iv0, jv1, iv1, msg, mA, mud, v_vmem,
             semA, semB, semC, semD, semI, semO):
    cid = jax.lax.axis_index("c")
    sid = jax.lax.axis_index("s")
    wid = cid * 16 + sid
    ebase = wid * EPW
    mus = [mu0, mu1, mu2]
    vs = [v0, v1, v2]

    def zero_acc():
        @pl.loop(0, ZR)
        def _(r):
            for cc in range(0, C, 16):
                msg[r, pl.ds(cc, 16)] = jnp.zeros((16,), jnp.float32)

        @pl.loop(0, (N // ZR + 15) // 16)
        def _(k):
            idx = k * 16 + sid

            @pl.when(idx < N // ZR)
            def _():
                pltpu.sync_copy(msg, acc.at[pl.ds(idx * ZR, ZR)])
        plsc.subcore_barrier()

    def flush(dst):
        plsc.subcore_barrier()

        @pl.when(sid == 0)
        def _():
            pltpu.sync_copy(acc, dst)
        plsc.subcore_barrier()

    def fetch_idx(base, jdst, idst):
        cj = pltpu.async_copy(jj.at[pl.ds(base, BEK)], jdst, semI)
        ci = pltpu.async_copy(ii.at[pl.ds(base, BEK)], idst, semI)
        return cj, ci

    zero_acc()

    def phase0(xg, wb):
        p0 = fetch_idx(ebase, jv0, iv0)
        p0[0].wait()
        p0[1].wait()

        @pl.loop(0, EPW, step=2 * BEK)
        def _(eo):
            def block(base, nxt_base, jvk, ivk, jvn, ivn):
                c3 = pltpu.async_copy(x.at[jvk], xg, semA)
                c4 = pltpu.async_copy(wij.at[pl.ds(base, BEK)], wb, semB)
                pn = fetch_idx(nxt_base, jvn, ivn)
                c3.wait()
                c4.wait()

                @pl.loop(0, BEK)
                def _(b):
                    for cc in range(0, C, 16):
                        s0 = (b, pl.ds(cc, 16))
                        s1 = (b, pl.ds(C + cc, 16))
                        s2 = (b, pl.ds(2 * C + cc, 16))
                        msg[s0] = wb[s0] * xg[s0]
                        xg[s1] = wb[s1] * xg[s1]
                        xg[s2] = wb[s2] * xg[s2]

                c5 = pltpu.async_copy(
                    xg.at[pl.ds(0, BEK), pl.ds(C, C)],
                    m12h.at[pl.ds(base, BEK), pl.ds(0, C)], semO)
                c6 = pltpu.async_copy(
                    xg.at[pl.ds(0, BEK), pl.ds(2 * C, C)],
                    m12h.at[pl.ds(base, BEK), pl.ds(C, C)], semO)
                pltpu.sync_copy(msg, acc.at[ivk], add=True)
                pn[0].wait()
                pn[1].wait()
                return c5, c6

            baseA = ebase + eo
            baseB = baseA + BEK
            nxtA = jnp.minimum(baseA + 2 * BEK, ebase + EPW - BEK)
            wA = block(baseA, baseB, jv0, iv0, jv1, iv1)
            wA[0].wait()
            wA[1].wait()
            wB = block(baseB, nxtA, jv1, iv1, jv0, iv0)
            wB[0].wait()
            wB[1].wait()

    pl.run_scoped(phase0,
                  pltpu.VMEM((BEK, 3 * C), jnp.float32),
                  pltpu.VMEM((BEK, 3 * C), jnp.float32))
    flush(dqp.at[cid])

    def dir_phases(msgq, mAq, mudq, v_vmemq):
        sets = [(jv0, iv0, mud, mA, msg, v_vmem, semA, semB),
                (jv1, iv1, mudq, mAq, msgq, v_vmemq, semC, semD)]

        def fire_inputs(d, base, st):
            jvk, _, mudk, mAk, msgk, vvk, sX, sY = st
            cg = pltpu.async_copy(mus[d].at[jvk], mudk, sX)
            c1 = pltpu.async_copy(m12h.at[pl.ds(base, BEK), pl.ds(0, C)],
                                  mAk, sY)
            c2 = pltpu.async_copy(m12h.at[pl.ds(base, BEK), pl.ds(C, C)],
                                  msgk, sY)
            c3 = pltpu.async_copy(vs[d].at[pl.ds(base, BEK)], vvk, sY)
            return cg, c1, c2, c3

        def drain_inputs(d, st):
            _, _, mudk, mAk, msgk, vvk, sX, sY = st
            pltpu.make_async_copy(mus[d].at[pl.ds(0, BEK)], mudk, sX).wait()
            pltpu.make_async_copy(m12h.at[pl.ds(0, BEK), pl.ds(0, C)],
                                  mAk, sY).wait()
            pltpu.make_async_copy(m12h.at[pl.ds(0, BEK), pl.ds(C, C)],
                                  msgk, sY).wait()
            pltpu.make_async_copy(vs[d].at[pl.ds(0, BEK)], vvk, sY).wait()

        def drain_idx(jdst, idst):
            pltpu.make_async_copy(jj.at[pl.ds(0, BEK)], jdst, semI).wait()
            pltpu.make_async_copy(ii.at[pl.ds(0, BEK)], idst, semI).wait()

        def compute_scatter(st):
            _, ivk, mudk, mAk, msgk, vvk, _, _ = st

            @pl.loop(0, BEK)
            def _(b):
                b16 = jax.lax.broadcast(b, (16,))
                vv = plsc.load_gather(vvk, [b16])
                for cc in range(0, C, 16):
                    sl = (b, pl.ds(cc, 16))
                    msgk[sl] = mAk[sl] * vv + msgk[sl] * mudk[sl]

            pltpu.sync_copy(msgk, acc.at[ivk], add=True)

        for d in range(3):
            zero_acc()
            pltpu.sync_copy(jj.at[pl.ds(ebase, BEK)], jv0)
            pltpu.sync_copy(ii.at[pl.ds(ebase, BEK)], iv0)
            fire_inputs(d, ebase, sets[0])
            pltpu.sync_copy(jj.at[pl.ds(ebase + BEK, BEK)], jv1)
            pltpu.sync_copy(ii.at[pl.ds(ebase + BEK, BEK)], iv1)

            @pl.loop(0, EPW, step=2 * BEK)
            def _(eo):
                baseA = ebase + eo
                baseB = baseA + BEK

                @pl.when(eo > 0)
                def _():
                    drain_idx(jv1, iv1)
                fire_inputs(d, baseB, sets[1])
                drain_inputs(d, sets[0])
                compute_scatter(sets[0])

                @pl.when(eo < EPW - 2 * BEK)
                def _():
                    fetch_idx(baseA + 2 * BEK, jv0, iv0)

                @pl.when(eo < EPW - 2 * BEK)
                def _():
                    drain_idx(jv0, iv0)
                    fire_inputs(d, baseA + 2 * BEK, sets[0])
                drain_inputs(d, sets[1])
                compute_scatter(sets[1])

                @pl.when(eo < EPW - 3 * BEK)
                def _():
                    fetch_idx(baseA + 3 * BEK, jv1, iv1)

            flush(dmup.at[d, cid])

    pl.run_scoped(dir_phases,
                  pltpu.VMEM((BEK, C), jnp.float32),
                  pltpu.VMEM((BEK, C), jnp.float32),
                  pltpu.VMEM((BEK, C), jnp.float32),
                  pltpu.VMEM((BEK,), jnp.float32))


def _sc_edge(x, wij, mu0, mu1, mu2, ii, jj, v0, v1, v2):
    mesh = plsc.VectorSubcoreMesh(core_axis_name="c", subcore_axis_name="s")
    f32 = jnp.float32
    cp = pltpu.CompilerParams()
    if "needs_layout_passes" in pltpu.CompilerParams.__dataclass_fields__:
        cp = dataclasses.replace(cp, needs_layout_passes=False)
    run = pl.kernel(
        _sc_body,
        mesh=mesh,
        compiler_params=cp,
        out_type=[
            jax.ShapeDtypeStruct((2, N, C), f32),
            jax.ShapeDtypeStruct((3, 2, N, C), f32),
            jax.ShapeDtypeStruct((E, 2 * C), f32),
        ],
        scratch_types=[
            pltpu.VMEM_SHARED((N, C), f32),
            pltpu.VMEM((BEK,), jnp.int32),
            pltpu.VMEM((BEK,), jnp.int32),
            pltpu.VMEM((BEK,), jnp.int32),
            pltpu.VMEM((BEK,), jnp.int32),
            pltpu.VMEM((BEK, C), f32),
            pltpu.VMEM((BEK, C), f32),
            pltpu.VMEM((BEK, C), f32),
            pltpu.VMEM((BEK,), f32),
            pltpu.SemaphoreType.DMA,
            pltpu.SemaphoreType.DMA,
            pltpu.SemaphoreType.DMA,
            pltpu.SemaphoreType.DMA,
            pltpu.SemaphoreType.DMA,
            pltpu.SemaphoreType.DMA,
        ],
    )
    return run(x, wij, mu0, mu1, mu2, ii, jj, v0, v1, v2)


def _mixing_body(q_ref, mu0_ref, mu1_ref, mu2_ref,
                 dq0_ref, dq1_ref,
                 dm00_ref, dm01_ref, dm10_ref, dm11_ref, dm20_ref, dm21_ref,
                 wmix_ref, wm1_ref, bm1_ref, wm2_ref, bm2_ref,
                 qo_ref, mo0_ref, mo1_ref, mo2_ref):
    qq = q_ref[...] + dq0_ref[...] + dq1_ref[...]
    mu2 = [mu0_ref[...] + dm00_ref[...] + dm01_ref[...],
           mu1_ref[...] + dm10_ref[...] + dm11_ref[...],
           mu2_ref[...] + dm20_ref[...] + dm21_ref[...]]
    wmix = wmix_ref[...]
    mix = [jnp.dot(m, wmix, preferred_element_type=jnp.float32) for m in mu2]
    muV = [m[:, :C] for m in mix]
    muW = [m[:, C:] for m in mix]
    muVn = jnp.sqrt(muV[0] * muV[0] + muV[1] * muV[1] + muV[2] * muV[2] + EPS)
    ctx = jnp.concatenate([qq, muVn], axis=1)
    h = _silu(jnp.dot(ctx, wm1_ref[...], preferred_element_type=jnp.float32)
              + bm1_ref[...])
    y = (jnp.dot(h, wm2_ref[...], preferred_element_type=jnp.float32)
         + bm2_ref[...])
    dq_i = y[:, :C]
    dmu_i = y[:, C:2 * C]
    dqmu_i = y[:, 2 * C:]
    s = muV[0] * muW[0] + muV[1] * muW[1] + muV[2] * muW[2]
    qo_ref[...] = qq + dq_i + dqmu_i * s
    mo0_ref[...] = mu2[0] + dmu_i * muW[0]
    mo1_ref[...] = mu2[1] + dmu_i * muW[1]
    mo2_ref[...] = mu2[2] + dmu_i * muW[2]


def _mixing(q, mu0, mu1, mu2, dq0, dq1, dm00, dm01, dm10, dm11, dm20, dm21,
            Wmix, Wm1, bm1, Wm2, bm2):
    node_spec = pl.BlockSpec((BN, C), lambda i: (i, 0))
    return pl.pallas_call(
        _mixing_body,
        grid=(N // BN,),
        in_specs=[node_spec] * 12 + [
            pl.BlockSpec((C, 2 * C), lambda i: (0, 0)),
            pl.BlockSpec((2 * C, C), lambda i: (0, 0)),
            pl.BlockSpec((1, C), lambda i: (0, 0)),
            pl.BlockSpec((C, 3 * C), lambda i: (0, 0)),
            pl.BlockSpec((1, 3 * C), lambda i: (0, 0)),
        ],
        out_specs=[node_spec, node_spec, node_spec, node_spec],
        out_shape=[jax.ShapeDtypeStruct((N, C), jnp.float32)] * 4,
    )(q, mu0, mu1, mu2, dq0, dq1, dm00, dm01, dm10, dm11, dm20, dm21,
      Wmix, Wm1, bm1.reshape(1, C), Wm2, bm2.reshape(1, 3 * C))


def kernel(q, mu, receivers, edge_indices, edge_weights, edge_versors, edge_attrs,
           W1, b1, W2, b2, Wf, bf, Wmix, Wm1, bm1, Wm2, bm2):
    del receivers
    x = _node_mlp(q, W1, b1, W2, b2)
    wij = _edge_filter(edge_attrs, edge_weights, Wf, bf)

    idx_i = edge_indices[0]
    idx_j = edge_indices[1]
    mu_d = [mu[:, d, :] for d in range(3)]
    v_d = [edge_versors[:, d] for d in range(3)]

    dqp, dmup, _m12 = _sc_edge(x, wij, mu_d[0], mu_d[1], mu_d[2],
                               idx_i, idx_j, v_d[0], v_d[1], v_d[2])

    qo, mo0, mo1, mo2 = _mixing(
        q, mu_d[0], mu_d[1], mu_d[2],
        dqp[0], dqp[1],
        dmup[0, 0], dmup[0, 1], dmup[1, 0], dmup[1, 1], dmup[2, 0], dmup[2, 1],
        Wmix, Wm1, bm1, Wm2, bm2)
    return qo, jnp.stack([mo0, mo1, mo2], axis=1)
